# baseline (speedup 1.0000x reference)
import functools

import jax
import jax.numpy as jnp
from jax import lax
from jax.experimental import pallas as pl
from jax.experimental.pallas import tpu as pltpu
from jax.experimental.pallas import tpu_sc as plsc

N = 10000
E = 320000
D = 128
H = 128
O = 64
G = 64
EPS = 0.3
BN_EPS = 1e-5

NUM_CORES = 2
NUM_SUBCORES = 16
NW = NUM_CORES * NUM_SUBCORES

CH = 50
NCHUNKS = E // CH
IB = 8
NR = NCHUNKS // (NW * IB)
ZR = 40
NZCH = N // ZR
NBUF = 4
G_AHEAD = 1
S_BEHIND = NBUF - G_AHEAD


def _sc_agg(h, src, dst):
  mesh = plsc.VectorSubcoreMesh(
      core_axis_name="c", subcore_axis_name="s",
      num_cores=NUM_CORES, num_subcores=NUM_SUBCORES)

  @functools.partial(
      pl.kernel,
      out_type=jax.ShapeDtypeStruct((2 * N, D), jnp.float32),
      mesh=mesh,
      scratch_types=(
          [pltpu.VMEM((2, IB, CH), jnp.int32)] * 2
          + [pltpu.VMEM((CH, D), jnp.float32)] * NBUF
          + [pltpu.VMEM_SHARED((N, D), jnp.float32)]
          + [pltpu.SemaphoreType.DMA] * 4
      ),
  )
  def agg_kernel(h_hbm, src_hbm, dst_hbm, out_hbm, *scratch):
    src_b, dst_b = scratch[0], scratch[1]
    rows = scratch[2:2 + NBUF]
    agg_sh = scratch[2 + NBUF]
    gsem, ssem, isem, wsem = scratch[3 + NBUF:7 + NBUF]
    cid = lax.axis_index("c")
    sid = lax.axis_index("s")
    wid = sid * NUM_CORES + cid
    rows0, rows1 = rows[0], rows[1]

    @pl.loop(0, ZR)
    def _(j):
      @pl.loop(0, D // 16)
      def _(k):
        rows0[j, pl.ds(k * 16, 16)] = jnp.zeros((16,), jnp.float32)

    zsrc = rows0.at[pl.ds(0, ZR)]

    @pl.loop(sid, NZCH, step=NUM_SUBCORES)
    def _(r):
      pltpu.sync_copy(zsrc, agg_sh.at[pl.ds(r * ZR, ZR)])

    plsc.subcore_barrier()

    def rbase(r):
      return (r * NW + wid) * IB

    def start_g(slot, j):
      pltpu.async_copy(h_hbm.at[src_b.at[slot, j]], rows[j % NBUF], gsem)

    def wait_g(slot, j):
      pltpu.make_async_copy(
          h_hbm.at[src_b.at[slot, j]], rows[j % NBUF], gsem).wait()

    def start_s(slot, j):
      pltpu.async_copy(rows[j % NBUF], agg_sh.at[dst_b.at[slot, j]], ssem,
                       add=True)

    def wait_s(slot, j):
      pltpu.make_async_copy(
          rows[j % NBUF], agg_sh.at[dst_b.at[slot, j]], ssem).wait()

    pltpu.sync_copy(src_hbm.at[pl.ds(rbase(0), IB)], src_b.at[0])
    pltpu.sync_copy(dst_hbm.at[pl.ds(rbase(0), IB)], dst_b.at[0])
    for j in range(G_AHEAD):
      start_g(0, j)

    @pl.loop(0, NR)
    def _(r):
      slot = lax.rem(r, 2)
      pslot = 1 - slot
      for j in range(IB):
        wait_g(slot, j)
        sb = j - S_BEHIND
        if sb < 0:
          @pl.when(r > 0)
          def _():
            wait_s(pslot, IB + sb)
        else:
          wait_s(slot, sb)
        if j == 2:
          @pl.when(r + 1 < NR)
          def _():
            nb = rbase(r + 1)
            pltpu.async_copy(src_hbm.at[pl.ds(nb, IB)], src_b.at[pslot], isem)
            pltpu.async_copy(dst_hbm.at[pl.ds(nb, IB)], dst_b.at[pslot], isem)
        ga = j + G_AHEAD
        if ga < IB:
          start_g(slot, ga)
        else:
          if ga == IB:
            @pl.when(r + 1 < NR)
            def _():
              pltpu.make_async_copy(
                  src_hbm.at[pl.ds(0, IB)], src_b.at[pslot], isem).wait()
              pltpu.make_async_copy(
                  dst_hbm.at[pl.ds(0, IB)], dst_b.at[pslot], isem).wait()

          @pl.when(r + 1 < NR)
          def _():
            start_g(pslot, ga - IB)
        start_s(slot, j)

    last_slot = (NR - 1) % 2
    for k in range(S_BEHIND):
      wait_s(last_slot, IB - S_BEHIND + k)

    plsc.subcore_barrier()

    wbufs = (rows0.at[pl.ds(0, ZR)], rows1.at[pl.ds(0, ZR)])
    nwsteps = (NZCH + NUM_SUBCORES - 1) // NUM_SUBCORES
    for t in range(nwsteps):
      r = sid + t * NUM_SUBCORES

      @pl.when(r < NZCH)
      def _():
        if t >= 2:
          pltpu.make_async_copy(
              wbufs[t % 2], out_hbm.at[pl.ds(cid * N, ZR)], wsem).wait()
        pltpu.sync_copy(agg_sh.at[pl.ds(r * ZR, ZR)], wbufs[t % 2])
        pltpu.async_copy(
            wbufs[t % 2], out_hbm.at[pl.ds(cid * N + r * ZR, ZR)], wsem)

    for t in range(nwsteps - 2, nwsteps):
      r = sid + t * NUM_SUBCORES

      @pl.when(r < NZCH)
      def _():
        pltpu.make_async_copy(
            wbufs[t % 2], out_hbm.at[pl.ds(cid * N, ZR)], wsem).wait()

  return agg_kernel(h, src, dst).reshape(2, N, D)


ROWS_BLK = 1000
NBLK = N // ROWS_BLK


def _row0(p, i):
  return jnp.where(p == 0, i, 0)


def _row1(p, i):
  return jnp.where(p == 1, i, 0)


def _tc_layer(h, parts, w1, b1, g, be, w2, b2):
  def body(h_ref, p_ref, w1_ref, b1_ref, g_ref, be_ref, w2_ref, b2_ref,
           out_ref, z_sc, ssum_sc, ssq_sc):
    p = pl.program_id(0)
    i = pl.program_id(1)

    @pl.when(p == 0)
    def _():
      pre = (1.0 + EPS) * h_ref[...] + p_ref[0] + p_ref[1]
      z = (jnp.dot(pre, w1_ref[...], preferred_element_type=jnp.float32)
           + b1_ref[...])
      z_sc[pl.ds(i * ROWS_BLK, ROWS_BLK), :] = z

      @pl.when(i == 0)
      def _():
        ssum_sc[...] = jnp.zeros_like(ssum_sc)
        ssq_sc[...] = jnp.zeros_like(ssq_sc)

      ssum_sc[...] += jnp.sum(z, axis=0, keepdims=True)
      ssq_sc[...] += jnp.sum(z * z, axis=0, keepdims=True)

    @pl.when(p == 1)
    def _():
      mean = ssum_sc[...] * (1.0 / N)
      var = ssq_sc[...] * (1.0 / N) - mean * mean
      scale = lax.rsqrt(var + BN_EPS) * g_ref[...]
      z = z_sc[pl.ds(i * ROWS_BLK, ROWS_BLK), :]
      t = jnp.tanh((z - mean) * scale + be_ref[...])
      out_ref[...] = jnp.tanh(
          jnp.dot(t, w2_ref[...], preferred_element_type=jnp.float32)
          + b2_ref[...])

  return pl.pallas_call(
      body,
      grid=(2, NBLK),
      in_specs=[
          pl.BlockSpec((ROWS_BLK, D), lambda p, i: (_row0(p, i), 0)),
          pl.BlockSpec((2, ROWS_BLK, D), lambda p, i: (0, _row0(p, i), 0)),
          pl.BlockSpec((D, H), lambda p, i: (0, 0)),
          pl.BlockSpec((1, H), lambda p, i: (0, 0)),
          pl.BlockSpec((1, H), lambda p, i: (0, 0)),
          pl.BlockSpec((1, H), lambda p, i: (0, 0)),
          pl.BlockSpec((H, H), lambda p, i: (0, 0)),
          pl.BlockSpec((1, H), lambda p, i: (0, 0)),
      ],
      out_specs=pl.BlockSpec((ROWS_BLK, H), lambda p, i: (_row1(p, i), 0)),
      out_shape=jax.ShapeDtypeStruct((N, H), jnp.float32),
      scratch_shapes=[
          pltpu.VMEM((N, H), jnp.float32),
          pltpu.VMEM((1, H), jnp.float32),
          pltpu.VMEM((1, H), jnp.float32),
      ],
  )(h, parts, w1, b1, g, be, w2, b2)


def _tc_layer_pool(h, parts, batch3, w1, b1, g, be, w2, b2, wl, bl):
  def body(h_ref, p_ref, b3_ref, w1_ref, b1_ref, g_ref, be_ref, w2_ref,
           b2_ref, wl_ref, bl_ref, out_ref, z_sc, ssum_sc, ssq_sc, pool_sc):
    p = pl.program_id(0)
    i = pl.program_id(1)

    @pl.when(p == 0)
    def _():
      pre = (1.0 + EPS) * h_ref[...] + p_ref[0] + p_ref[1]
      z = (jnp.dot(pre, w1_ref[...], preferred_element_type=jnp.float32)
           + b1_ref[...])
      z_sc[pl.ds(i * ROWS_BLK, ROWS_BLK), :] = z

      @pl.when(i == 0)
      def _():
        ssum_sc[...] = jnp.zeros_like(ssum_sc)
        ssq_sc[...] = jnp.zeros_like(ssq_sc)

      ssum_sc[...] += jnp.sum(z, axis=0, keepdims=True)
      ssq_sc[...] += jnp.sum(z * z, axis=0, keepdims=True)

    @pl.when(p == 1)
    def _():
      mean = ssum_sc[...] * (1.0 / N)
      var = ssq_sc[...] * (1.0 / N) - mean * mean
      scale = lax.rsqrt(var + BN_EPS) * g_ref[...]
      z = z_sc[pl.ds(i * ROWS_BLK, ROWS_BLK), :]
      t = jnp.tanh((z - mean) * scale + be_ref[...])
      h2 = jnp.tanh(
          jnp.dot(t, w2_ref[...], preferred_element_type=jnp.float32)
          + b2_ref[...])

      @pl.when(i == 0)
      def _():
        pool_sc[...] = jnp.zeros_like(pool_sc)

      bb = b3_ref[0, 0, :]
      oh = (bb[:, None] == lax.broadcasted_iota(jnp.int32, (ROWS_BLK, G), 1)
            ).astype(jnp.float32)
      pool_sc[...] += lax.dot_general(
          oh, h2, (((0,), (0,)), ((), ())),
          preferred_element_type=jnp.float32)

      @pl.when(i == NBLK - 1)
      def _():
        out_ref[...] = jax.nn.sigmoid(
            jnp.dot(pool_sc[...], wl_ref[...],
                    preferred_element_type=jnp.float32) + bl_ref[...])

  return pl.pallas_call(
      body,
      grid=(2, NBLK),
      in_specs=[
          pl.BlockSpec((ROWS_BLK, D), lambda p, i: (_row0(p, i), 0)),
          pl.BlockSpec((2, ROWS_BLK, D), lambda p, i: (0, _row0(p, i), 0)),
          pl.BlockSpec((1, 1, ROWS_BLK), lambda p, i: (_row1(p, i), 0, 0)),
          pl.BlockSpec((D, H), lambda p, i: (0, 0)),
          pl.BlockSpec((1, H), lambda p, i: (0, 0)),
          pl.BlockSpec((1, H), lambda p, i: (0, 0)),
          pl.BlockSpec((1, H), lambda p, i: (0, 0)),
          pl.BlockSpec((H, H), lambda p, i: (0, 0)),
          pl.BlockSpec((1, H), lambda p, i: (0, 0)),
          pl.BlockSpec((H, O), lambda p, i: (0, 0)),
          pl.BlockSpec((1, O), lambda p, i: (0, 0)),
      ],
      out_specs=pl.BlockSpec((G, O), lambda p, i: (0, 0)),
      out_shape=jax.ShapeDtypeStruct((G, O), jnp.float32),
      scratch_shapes=[
          pltpu.VMEM((N, H), jnp.float32),
          pltpu.VMEM((1, H), jnp.float32),
          pltpu.VMEM((1, H), jnp.float32),
          pltpu.VMEM((G, H), jnp.float32),
      ],
  )(h, parts, batch3, w1, b1, g, be, w2, b2, wl, bl)


def kernel(x, edge_index, batch,
           W1_0, b1_0, g_0, be_0, W2_0, b2_0,
           W1_1, b1_1, g_1, be_1, W2_1, b2_1,
           W1_2, b1_2, g_2, be_2, W2_2, b2_2,
           Wl, bl):
  src = edge_index[0].reshape(NCHUNKS, CH)
  dst = edge_index[1].reshape(NCHUNKS, CH)
  batch3 = batch.reshape(NBLK, 1, ROWS_BLK)
  layer_params = [
      (W1_0, b1_0, g_0, be_0, W2_0, b2_0),
      (W1_1, b1_1, g_1, be_1, W2_1, b2_1),
      (W1_2, b1_2, g_2, be_2, W2_2, b2_2),
  ]
  h = x
  for li, (w1, b1, g, be, w2, b2) in enumerate(layer_params):
    parts = _sc_agg(h, src, dst)
    if li < 2:
      h = _tc_layer(h, parts, w1, b1.reshape(1, H), g.reshape(1, H),
                    be.reshape(1, H), w2, b2.reshape(1, H))
    else:
      return _tc_layer_pool(h, parts, batch3, w1, b1.reshape(1, H),
                            g.reshape(1, H), be.reshape(1, H), w2,
                            b2.reshape(1, H), Wl, bl.reshape(1, O))


# speedup vs baseline: 1.8404x; 1.8404x over previous
import functools

import jax
import jax.numpy as jnp
from jax import lax
from jax.experimental import pallas as pl
from jax.experimental.pallas import tpu as pltpu
from jax.experimental.pallas import tpu_sc as plsc

N = 10000
E = 320000
D = 128
H = 128
O = 64
G = 64
EPS = 0.3
BN_EPS = 1e-5

NUM_CORES = 2
NUM_SUBCORES = 16
NW = NUM_CORES * NUM_SUBCORES

CH = 50
NCHUNKS = E // CH
IB = 8
NR = NCHUNKS // (NW * IB)
ZR = 40
NZCH = N // ZR
NBUF = 4
G_AHEAD = 3
S_BEHIND = NBUF - G_AHEAD


def _sc_agg(h, src, dst):
  mesh = plsc.VectorSubcoreMesh(
      core_axis_name="c", subcore_axis_name="s",
      num_cores=NUM_CORES, num_subcores=NUM_SUBCORES)

  @functools.partial(
      pl.kernel,
      out_type=jax.ShapeDtypeStruct((2 * N, D), jnp.float32),
      mesh=mesh,
      scratch_types=(
          [pltpu.VMEM((2, IB, CH), jnp.int32)] * 2
          + [pltpu.VMEM((CH, D), jnp.float32)] * NBUF
          + [pltpu.VMEM_SHARED((N, D), jnp.float32)]
          + [pltpu.SemaphoreType.DMA] * 4
      ),
  )
  def agg_kernel(h_hbm, src_hbm, dst_hbm, out_hbm, *scratch):
    src_b, dst_b = scratch[0], scratch[1]
    rows = scratch[2:2 + NBUF]
    agg_sh = scratch[2 + NBUF]
    gsem, ssem, isem, wsem = scratch[3 + NBUF:7 + NBUF]
    cid = lax.axis_index("c")
    sid = lax.axis_index("s")
    wid = sid * NUM_CORES + cid
    rows0, rows1 = rows[0], rows[1]

    @pl.loop(0, ZR)
    def _(j):
      @pl.loop(0, D // 16)
      def _(k):
        rows0[j, pl.ds(k * 16, 16)] = jnp.zeros((16,), jnp.float32)

    zsrc = rows0.at[pl.ds(0, ZR)]

    @pl.loop(sid, NZCH, step=NUM_SUBCORES)
    def _(r):
      pltpu.sync_copy(zsrc, agg_sh.at[pl.ds(r * ZR, ZR)])

    plsc.subcore_barrier()

    def rbase(r):
      return (r * NW + wid) * IB

    def start_g(slot, j):
      pltpu.async_copy(h_hbm.at[src_b.at[slot, j]], rows[j % NBUF], gsem)

    def wait_g(slot, j):
      pltpu.make_async_copy(
          h_hbm.at[src_b.at[slot, j]], rows[j % NBUF], gsem).wait()

    def start_s(slot, j):
      pltpu.async_copy(rows[j % NBUF], agg_sh.at[dst_b.at[slot, j]], ssem,
                       add=True)

    def wait_s(slot, j):
      pltpu.make_async_copy(
          rows[j % NBUF], agg_sh.at[dst_b.at[slot, j]], ssem).wait()

    pltpu.sync_copy(src_hbm.at[pl.ds(rbase(0), IB)], src_b.at[0])
    pltpu.sync_copy(dst_hbm.at[pl.ds(rbase(0), IB)], dst_b.at[0])
    for j in range(G_AHEAD):
      start_g(0, j)

    @pl.loop(0, NR)
    def _(r):
      slot = lax.rem(r, 2)
      pslot = 1 - slot
      for j in range(IB):
        wait_g(slot, j)
        sb = j - S_BEHIND
        if sb < 0:
          @pl.when(r > 0)
          def _():
            wait_s(pslot, IB + sb)
        else:
          wait_s(slot, sb)
        if j == 2:
          @pl.when(r + 1 < NR)
          def _():
            nb = rbase(r + 1)
            pltpu.async_copy(src_hbm.at[pl.ds(nb, IB)], src_b.at[pslot], isem)
            pltpu.async_copy(dst_hbm.at[pl.ds(nb, IB)], dst_b.at[pslot], isem)
        ga = j + G_AHEAD
        if ga < IB:
          start_g(slot, ga)
        else:
          if ga == IB:
            @pl.when(r + 1 < NR)
            def _():
              pltpu.make_async_copy(
                  src_hbm.at[pl.ds(0, IB)], src_b.at[pslot], isem).wait()
              pltpu.make_async_copy(
                  dst_hbm.at[pl.ds(0, IB)], dst_b.at[pslot], isem).wait()

          @pl.when(r + 1 < NR)
          def _():
            start_g(pslot, ga - IB)
        start_s(slot, j)

    last_slot = (NR - 1) % 2
    for k in range(S_BEHIND):
      wait_s(last_slot, IB - S_BEHIND + k)

    plsc.subcore_barrier()

    wbufs = (rows0.at[pl.ds(0, ZR)], rows1.at[pl.ds(0, ZR)])
    nwsteps = (NZCH + NUM_SUBCORES - 1) // NUM_SUBCORES
    for t in range(nwsteps):
      r = sid + t * NUM_SUBCORES

      @pl.when(r < NZCH)
      def _():
        if t >= 2:
          pltpu.make_async_copy(
              wbufs[t % 2], out_hbm.at[pl.ds(cid * N, ZR)], wsem).wait()
        pltpu.sync_copy(agg_sh.at[pl.ds(r * ZR, ZR)], wbufs[t % 2])
        pltpu.async_copy(
            wbufs[t % 2], out_hbm.at[pl.ds(cid * N + r * ZR, ZR)], wsem)

    for t in range(nwsteps - 2, nwsteps):
      r = sid + t * NUM_SUBCORES

      @pl.when(r < NZCH)
      def _():
        pltpu.make_async_copy(
            wbufs[t % 2], out_hbm.at[pl.ds(cid * N, ZR)], wsem).wait()

  return agg_kernel(h, src, dst).reshape(2, N, D)


ROWS_BLK = 1000
NBLK = N // ROWS_BLK


def _row0(p, i):
  return jnp.where(p == 0, i, 0)


def _row1(p, i):
  return jnp.where(p == 1, i, 0)


def _tc_layer(h, parts, w1, b1, g, be, w2, b2):
  def body(h_ref, p_ref, w1_ref, b1_ref, g_ref, be_ref, w2_ref, b2_ref,
           out_ref, z_sc, ssum_sc, ssq_sc):
    p = pl.program_id(0)
    i = pl.program_id(1)

    @pl.when(p == 0)
    def _():
      pre = (1.0 + EPS) * h_ref[...] + p_ref[0] + p_ref[1]
      z = (jnp.dot(pre, w1_ref[...], preferred_element_type=jnp.float32)
           + b1_ref[...])
      z_sc[pl.ds(i * ROWS_BLK, ROWS_BLK), :] = z

      @pl.when(i == 0)
      def _():
        ssum_sc[...] = jnp.zeros_like(ssum_sc)
        ssq_sc[...] = jnp.zeros_like(ssq_sc)

      ssum_sc[...] += jnp.sum(z, axis=0, keepdims=True)
      ssq_sc[...] += jnp.sum(z * z, axis=0, keepdims=True)

    @pl.when(p == 1)
    def _():
      mean = ssum_sc[...] * (1.0 / N)
      var = ssq_sc[...] * (1.0 / N) - mean * mean
      scale = lax.rsqrt(var + BN_EPS) * g_ref[...]
      z = z_sc[pl.ds(i * ROWS_BLK, ROWS_BLK), :]
      t = jnp.tanh((z - mean) * scale + be_ref[...])
      out_ref[...] = jnp.tanh(
          jnp.dot(t, w2_ref[...], preferred_element_type=jnp.float32)
          + b2_ref[...])

  return pl.pallas_call(
      body,
      grid=(2, NBLK),
      in_specs=[
          pl.BlockSpec((ROWS_BLK, D), lambda p, i: (_row0(p, i), 0)),
          pl.BlockSpec((2, ROWS_BLK, D), lambda p, i: (0, _row0(p, i), 0)),
          pl.BlockSpec((D, H), lambda p, i: (0, 0)),
          pl.BlockSpec((1, H), lambda p, i: (0, 0)),
          pl.BlockSpec((1, H), lambda p, i: (0, 0)),
          pl.BlockSpec((1, H), lambda p, i: (0, 0)),
          pl.BlockSpec((H, H), lambda p, i: (0, 0)),
          pl.BlockSpec((1, H), lambda p, i: (0, 0)),
      ],
      out_specs=pl.BlockSpec((ROWS_BLK, H), lambda p, i: (_row1(p, i), 0)),
      out_shape=jax.ShapeDtypeStruct((N, H), jnp.float32),
      scratch_shapes=[
          pltpu.VMEM((N, H), jnp.float32),
          pltpu.VMEM((1, H), jnp.float32),
          pltpu.VMEM((1, H), jnp.float32),
      ],
  )(h, parts, w1, b1, g, be, w2, b2)


def _tc_layer_pool(h, parts, batch3, w1, b1, g, be, w2, b2, wl, bl):
  def body(h_ref, p_ref, b3_ref, w1_ref, b1_ref, g_ref, be_ref, w2_ref,
           b2_ref, wl_ref, bl_ref, out_ref, z_sc, ssum_sc, ssq_sc, pool_sc):
    p = pl.program_id(0)
    i = pl.program_id(1)

    @pl.when(p == 0)
    def _():
      pre = (1.0 + EPS) * h_ref[...] + p_ref[0] + p_ref[1]
      z = (jnp.dot(pre, w1_ref[...], preferred_element_type=jnp.float32)
           + b1_ref[...])
      z_sc[pl.ds(i * ROWS_BLK, ROWS_BLK), :] = z

      @pl.when(i == 0)
      def _():
        ssum_sc[...] = jnp.zeros_like(ssum_sc)
        ssq_sc[...] = jnp.zeros_like(ssq_sc)

      ssum_sc[...] += jnp.sum(z, axis=0, keepdims=True)
      ssq_sc[...] += jnp.sum(z * z, axis=0, keepdims=True)

    @pl.when(p == 1)
    def _():
      mean = ssum_sc[...] * (1.0 / N)
      var = ssq_sc[...] * (1.0 / N) - mean * mean
      scale = lax.rsqrt(var + BN_EPS) * g_ref[...]
      z = z_sc[pl.ds(i * ROWS_BLK, ROWS_BLK), :]
      t = jnp.tanh((z - mean) * scale + be_ref[...])
      h2 = jnp.tanh(
          jnp.dot(t, w2_ref[...], preferred_element_type=jnp.float32)
          + b2_ref[...])

      @pl.when(i == 0)
      def _():
        pool_sc[...] = jnp.zeros_like(pool_sc)

      bb = b3_ref[0, 0, :]
      oh = (bb[:, None] == lax.broadcasted_iota(jnp.int32, (ROWS_BLK, G), 1)
            ).astype(jnp.float32)
      pool_sc[...] += lax.dot_general(
          oh, h2, (((0,), (0,)), ((), ())),
          preferred_element_type=jnp.float32)

      @pl.when(i == NBLK - 1)
      def _():
        out_ref[...] = jax.nn.sigmoid(
            jnp.dot(pool_sc[...], wl_ref[...],
                    preferred_element_type=jnp.float32) + bl_ref[...])

  return pl.pallas_call(
      body,
      grid=(2, NBLK),
      in_specs=[
          pl.BlockSpec((ROWS_BLK, D), lambda p, i: (_row0(p, i), 0)),
          pl.BlockSpec((2, ROWS_BLK, D), lambda p, i: (0, _row0(p, i), 0)),
          pl.BlockSpec((1, 1, ROWS_BLK), lambda p, i: (_row1(p, i), 0, 0)),
          pl.BlockSpec((D, H), lambda p, i: (0, 0)),
          pl.BlockSpec((1, H), lambda p, i: (0, 0)),
          pl.BlockSpec((1, H), lambda p, i: (0, 0)),
          pl.BlockSpec((1, H), lambda p, i: (0, 0)),
          pl.BlockSpec((H, H), lambda p, i: (0, 0)),
          pl.BlockSpec((1, H), lambda p, i: (0, 0)),
          pl.BlockSpec((H, O), lambda p, i: (0, 0)),
          pl.BlockSpec((1, O), lambda p, i: (0, 0)),
      ],
      out_specs=pl.BlockSpec((G, O), lambda p, i: (0, 0)),
      out_shape=jax.ShapeDtypeStruct((G, O), jnp.float32),
      scratch_shapes=[
          pltpu.VMEM((N, H), jnp.float32),
          pltpu.VMEM((1, H), jnp.float32),
          pltpu.VMEM((1, H), jnp.float32),
          pltpu.VMEM((G, H), jnp.float32),
      ],
  )(h, parts, batch3, w1, b1, g, be, w2, b2, wl, bl)


def kernel(x, edge_index, batch,
           W1_0, b1_0, g_0, be_0, W2_0, b2_0,
           W1_1, b1_1, g_1, be_1, W2_1, b2_1,
           W1_2, b1_2, g_2, be_2, W2_2, b2_2,
           Wl, bl):
  src = edge_index[0].reshape(NCHUNKS, CH)
  dst = edge_index[1].reshape(NCHUNKS, CH)
  batch3 = batch.reshape(NBLK, 1, ROWS_BLK)
  layer_params = [
      (W1_0, b1_0, g_0, be_0, W2_0, b2_0),
      (W1_1, b1_1, g_1, be_1, W2_1, b2_1),
      (W1_2, b1_2, g_2, be_2, W2_2, b2_2),
  ]
  h = x
  for li, (w1, b1, g, be, w2, b2) in enumerate(layer_params):
    parts = _sc_agg(h, src, dst)
    if li < 2:
      h = _tc_layer(h, parts, w1, b1.reshape(1, H), g.reshape(1, H),
                    be.reshape(1, H), w2, b2.reshape(1, H))
    else:
      return _tc_layer_pool(h, parts, batch3, w1, b1.reshape(1, H),
                            g.reshape(1, H), be.reshape(1, H), w2,
                            b2.reshape(1, H), Wl, bl.reshape(1, O))


# speedup vs baseline: 1.8419x; 1.0008x over previous
import functools

import jax
import jax.numpy as jnp
from jax import lax
from jax.experimental import pallas as pl
from jax.experimental.pallas import tpu as pltpu
from jax.experimental.pallas import tpu_sc as plsc

N = 10000
E = 320000
D = 128
H = 128
O = 64
G = 64
EPS = 0.3
BN_EPS = 1e-5

NUM_CORES = 2
NUM_SUBCORES = 16
NW = NUM_CORES * NUM_SUBCORES

CH = 50
NCHUNKS = E // CH
IB = 8
NR = NCHUNKS // (NW * IB)
ZR = 40
NZCH = N // ZR
NBUF = 4
G_AHEAD = 3
S_BEHIND = NBUF - G_AHEAD
PREFETCH_J = 0 if IB - G_AHEAD <= 2 else 2


def _sc_agg(h, src, dst):
  mesh = plsc.VectorSubcoreMesh(
      core_axis_name="c", subcore_axis_name="s",
      num_cores=NUM_CORES, num_subcores=NUM_SUBCORES)

  @functools.partial(
      pl.kernel,
      out_type=jax.ShapeDtypeStruct((2 * N, D), jnp.float32),
      mesh=mesh,
      scratch_types=(
          [pltpu.VMEM((2, IB, CH), jnp.int32)] * 2
          + [pltpu.VMEM((CH, D), jnp.float32)] * NBUF
          + [pltpu.VMEM((ZR, D), jnp.float32)] * 2
          + [pltpu.VMEM_SHARED((N, D), jnp.float32)]
          + [pltpu.SemaphoreType.DMA] * 4
      ),
  )
  def agg_kernel(h_hbm, src_hbm, dst_hbm, out_hbm, *scratch):
    src_b, dst_b = scratch[0], scratch[1]
    rows = scratch[2:2 + NBUF]
    zb0, zb1 = scratch[2 + NBUF], scratch[3 + NBUF]
    agg_sh = scratch[4 + NBUF]
    gsem, ssem, isem, wsem = scratch[5 + NBUF:9 + NBUF]
    cid = lax.axis_index("c")
    sid = lax.axis_index("s")
    wid = sid * NUM_CORES + cid

    @pl.loop(0, ZR)
    def _(j):
      @pl.loop(0, D // 16)
      def _(k):
        zb0[j, pl.ds(k * 16, 16)] = jnp.zeros((16,), jnp.float32)

    @pl.loop(sid, NZCH, step=NUM_SUBCORES)
    def _(r):
      pltpu.sync_copy(zb0, agg_sh.at[pl.ds(r * ZR, ZR)])

    plsc.subcore_barrier()

    def rbase(r):
      return (r * NW + wid) * IB

    def start_g(slot, j):
      pltpu.async_copy(h_hbm.at[src_b.at[slot, j]], rows[j % NBUF], gsem)

    def wait_g(slot, j):
      pltpu.make_async_copy(
          h_hbm.at[src_b.at[slot, j]], rows[j % NBUF], gsem).wait()

    def start_s(slot, j):
      pltpu.async_copy(rows[j % NBUF], agg_sh.at[dst_b.at[slot, j]], ssem,
                       add=True)

    def wait_s(slot, j):
      pltpu.make_async_copy(
          rows[j % NBUF], agg_sh.at[dst_b.at[slot, j]], ssem).wait()

    pltpu.sync_copy(src_hbm.at[pl.ds(rbase(0), IB)], src_b.at[0])
    pltpu.sync_copy(dst_hbm.at[pl.ds(rbase(0), IB)], dst_b.at[0])
    for j in range(G_AHEAD):
      start_g(0, j)

    @pl.loop(0, NR)
    def _(r):
      slot = lax.rem(r, 2)
      pslot = 1 - slot
      for j in range(IB):
        wait_g(slot, j)
        sb = j - S_BEHIND
        if sb < 0:
          @pl.when(r > 0)
          def _():
            wait_s(pslot, IB + sb)
        else:
          wait_s(slot, sb)
        if j == PREFETCH_J:
          @pl.when(r + 1 < NR)
          def _():
            nb = rbase(r + 1)
            pltpu.async_copy(src_hbm.at[pl.ds(nb, IB)], src_b.at[pslot], isem)
            pltpu.async_copy(dst_hbm.at[pl.ds(nb, IB)], dst_b.at[pslot], isem)
        ga = j + G_AHEAD
        if ga < IB:
          start_g(slot, ga)
        else:
          if ga == IB:
            @pl.when(r + 1 < NR)
            def _():
              pltpu.make_async_copy(
                  src_hbm.at[pl.ds(0, IB)], src_b.at[pslot], isem).wait()
              pltpu.make_async_copy(
                  dst_hbm.at[pl.ds(0, IB)], dst_b.at[pslot], isem).wait()

          @pl.when(r + 1 < NR)
          def _():
            start_g(pslot, ga - IB)
        start_s(slot, j)

    last_slot = (NR - 1) % 2
    for k in range(S_BEHIND):
      wait_s(last_slot, IB - S_BEHIND + k)

    plsc.subcore_barrier()

    wbufs = (zb0, zb1)
    nwsteps = (NZCH + NUM_SUBCORES - 1) // NUM_SUBCORES
    for t in range(nwsteps):
      r = sid + t * NUM_SUBCORES

      @pl.when(r < NZCH)
      def _():
        if t >= 2:
          pltpu.make_async_copy(
              wbufs[t % 2], out_hbm.at[pl.ds(cid * N, ZR)], wsem).wait()
        pltpu.sync_copy(agg_sh.at[pl.ds(r * ZR, ZR)], wbufs[t % 2])
        pltpu.async_copy(
            wbufs[t % 2], out_hbm.at[pl.ds(cid * N + r * ZR, ZR)], wsem)

    for t in range(nwsteps - 2, nwsteps):
      r = sid + t * NUM_SUBCORES

      @pl.when(r < NZCH)
      def _():
        pltpu.make_async_copy(
            wbufs[t % 2], out_hbm.at[pl.ds(cid * N, ZR)], wsem).wait()

  return agg_kernel(h, src, dst).reshape(2, N, D)


ROWS_BLK = 1000
NBLK = N // ROWS_BLK


def _row0(p, i):
  return jnp.where(p == 0, i, 0)


def _row1(p, i):
  return jnp.where(p == 1, i, 0)


def _tc_layer(h, parts, w1, b1, g, be, w2, b2):
  def body(h_ref, p_ref, w1_ref, b1_ref, g_ref, be_ref, w2_ref, b2_ref,
           out_ref, z_sc, ssum_sc, ssq_sc):
    p = pl.program_id(0)
    i = pl.program_id(1)

    @pl.when(p == 0)
    def _():
      pre = (1.0 + EPS) * h_ref[...] + p_ref[0] + p_ref[1]
      z = (jnp.dot(pre, w1_ref[...], preferred_element_type=jnp.float32)
           + b1_ref[...])
      z_sc[pl.ds(i * ROWS_BLK, ROWS_BLK), :] = z

      @pl.when(i == 0)
      def _():
        ssum_sc[...] = jnp.zeros_like(ssum_sc)
        ssq_sc[...] = jnp.zeros_like(ssq_sc)

      ssum_sc[...] += jnp.sum(z, axis=0, keepdims=True)
      ssq_sc[...] += jnp.sum(z * z, axis=0, keepdims=True)

    @pl.when(p == 1)
    def _():
      mean = ssum_sc[...] * (1.0 / N)
      var = ssq_sc[...] * (1.0 / N) - mean * mean
      scale = lax.rsqrt(var + BN_EPS) * g_ref[...]
      z = z_sc[pl.ds(i * ROWS_BLK, ROWS_BLK), :]
      t = jnp.tanh((z - mean) * scale + be_ref[...])
      out_ref[...] = jnp.tanh(
          jnp.dot(t, w2_ref[...], preferred_element_type=jnp.float32)
          + b2_ref[...])

  return pl.pallas_call(
      body,
      grid=(2, NBLK),
      in_specs=[
          pl.BlockSpec((ROWS_BLK, D), lambda p, i: (_row0(p, i), 0)),
          pl.BlockSpec((2, ROWS_BLK, D), lambda p, i: (0, _row0(p, i), 0)),
          pl.BlockSpec((D, H), lambda p, i: (0, 0)),
          pl.BlockSpec((1, H), lambda p, i: (0, 0)),
          pl.BlockSpec((1, H), lambda p, i: (0, 0)),
          pl.BlockSpec((1, H), lambda p, i: (0, 0)),
          pl.BlockSpec((H, H), lambda p, i: (0, 0)),
          pl.BlockSpec((1, H), lambda p, i: (0, 0)),
      ],
      out_specs=pl.BlockSpec((ROWS_BLK, H), lambda p, i: (_row1(p, i), 0)),
      out_shape=jax.ShapeDtypeStruct((N, H), jnp.float32),
      scratch_shapes=[
          pltpu.VMEM((N, H), jnp.float32),
          pltpu.VMEM((1, H), jnp.float32),
          pltpu.VMEM((1, H), jnp.float32),
      ],
  )(h, parts, w1, b1, g, be, w2, b2)


def _tc_layer_pool(h, parts, batch3, w1, b1, g, be, w2, b2, wl, bl):
  def body(h_ref, p_ref, b3_ref, w1_ref, b1_ref, g_ref, be_ref, w2_ref,
           b2_ref, wl_ref, bl_ref, out_ref, z_sc, ssum_sc, ssq_sc, pool_sc):
    p = pl.program_id(0)
    i = pl.program_id(1)

    @pl.when(p == 0)
    def _():
      pre = (1.0 + EPS) * h_ref[...] + p_ref[0] + p_ref[1]
      z = (jnp.dot(pre, w1_ref[...], preferred_element_type=jnp.float32)
           + b1_ref[...])
      z_sc[pl.ds(i * ROWS_BLK, ROWS_BLK), :] = z

      @pl.when(i == 0)
      def _():
        ssum_sc[...] = jnp.zeros_like(ssum_sc)
        ssq_sc[...] = jnp.zeros_like(ssq_sc)

      ssum_sc[...] += jnp.sum(z, axis=0, keepdims=True)
      ssq_sc[...] += jnp.sum(z * z, axis=0, keepdims=True)

    @pl.when(p == 1)
    def _():
      mean = ssum_sc[...] * (1.0 / N)
      var = ssq_sc[...] * (1.0 / N) - mean * mean
      scale = lax.rsqrt(var + BN_EPS) * g_ref[...]
      z = z_sc[pl.ds(i * ROWS_BLK, ROWS_BLK), :]
      t = jnp.tanh((z - mean) * scale + be_ref[...])
      h2 = jnp.tanh(
          jnp.dot(t, w2_ref[...], preferred_element_type=jnp.float32)
          + b2_ref[...])

      @pl.when(i == 0)
      def _():
        pool_sc[...] = jnp.zeros_like(pool_sc)

      bb = b3_ref[0, 0, :]
      oh = (bb[:, None] == lax.broadcasted_iota(jnp.int32, (ROWS_BLK, G), 1)
            ).astype(jnp.float32)
      pool_sc[...] += lax.dot_general(
          oh, h2, (((0,), (0,)), ((), ())),
          preferred_element_type=jnp.float32)

      @pl.when(i == NBLK - 1)
      def _():
        out_ref[...] = jax.nn.sigmoid(
            jnp.dot(pool_sc[...], wl_ref[...],
                    preferred_element_type=jnp.float32) + bl_ref[...])

  return pl.pallas_call(
      body,
      grid=(2, NBLK),
      in_specs=[
          pl.BlockSpec((ROWS_BLK, D), lambda p, i: (_row0(p, i), 0)),
          pl.BlockSpec((2, ROWS_BLK, D), lambda p, i: (0, _row0(p, i), 0)),
          pl.BlockSpec((1, 1, ROWS_BLK), lambda p, i: (_row1(p, i), 0, 0)),
          pl.BlockSpec((D, H), lambda p, i: (0, 0)),
          pl.BlockSpec((1, H), lambda p, i: (0, 0)),
          pl.BlockSpec((1, H), lambda p, i: (0, 0)),
          pl.BlockSpec((1, H), lambda p, i: (0, 0)),
          pl.BlockSpec((H, H), lambda p, i: (0, 0)),
          pl.BlockSpec((1, H), lambda p, i: (0, 0)),
          pl.BlockSpec((H, O), lambda p, i: (0, 0)),
          pl.BlockSpec((1, O), lambda p, i: (0, 0)),
      ],
      out_specs=pl.BlockSpec((G, O), lambda p, i: (0, 0)),
      out_shape=jax.ShapeDtypeStruct((G, O), jnp.float32),
      scratch_shapes=[
          pltpu.VMEM((N, H), jnp.float32),
          pltpu.VMEM((1, H), jnp.float32),
          pltpu.VMEM((1, H), jnp.float32),
          pltpu.VMEM((G, H), jnp.float32),
      ],
  )(h, parts, batch3, w1, b1, g, be, w2, b2, wl, bl)


def kernel(x, edge_index, batch,
           W1_0, b1_0, g_0, be_0, W2_0, b2_0,
           W1_1, b1_1, g_1, be_1, W2_1, b2_1,
           W1_2, b1_2, g_2, be_2, W2_2, b2_2,
           Wl, bl):
  src = edge_index[0].reshape(NCHUNKS, CH)
  dst = edge_index[1].reshape(NCHUNKS, CH)
  batch3 = batch.reshape(NBLK, 1, ROWS_BLK)
  layer_params = [
      (W1_0, b1_0, g_0, be_0, W2_0, b2_0),
      (W1_1, b1_1, g_1, be_1, W2_1, b2_1),
      (W1_2, b1_2, g_2, be_2, W2_2, b2_2),
  ]
  h = x
  for li, (w1, b1, g, be, w2, b2) in enumerate(layer_params):
    parts = _sc_agg(h, src, dst)
    if li < 2:
      h = _tc_layer(h, parts, w1, b1.reshape(1, H), g.reshape(1, H),
                    be.reshape(1, H), w2, b2.reshape(1, H))
    else:
      return _tc_layer_pool(h, parts, batch3, w1, b1.reshape(1, H),
                            g.reshape(1, H), be.reshape(1, H), w2,
                            b2.reshape(1, H), Wl, bl.reshape(1, O))
